# baseline (speedup 1.0000x reference)
import functools

import jax
import jax.numpy as jnp
from jax import lax
from jax.experimental import pallas as pl
from jax.experimental.pallas import tpu as pltpu
from jax.experimental.pallas import tpu_sc as plsc

_D = 768
_F = 12288
_K = 32
_CH = 128
_NCH = _F // _CH
_SUB = 32
_NSUBS = 4
_NP = 3328
_TB_MM = 832
_FB = 1024
_TB_SEL = 208
_GCAP = 64
_UNROLL = 32


def _enc_body(x_ref, w_ref, b_ref, o_ref):
    xc = (x_ref[...] - b_ref[...]).astype(jnp.bfloat16)
    a = jnp.maximum(
        jnp.dot(xc, w_ref[...], preferred_element_type=jnp.float32), 0.0)
    o_ref[...] = a.reshape(a.shape[0], _FB // _CH, _CH)


def _cmax_body(a_ref, s0_ref, s1_ref, s2_ref, s3_ref, c_ref):
    a = a_ref[...]
    subs = [jnp.max(a[:, :, s * _SUB:(s + 1) * _SUB], axis=2)
            for s in range(_NSUBS)]
    for ref, m in zip((s0_ref, s1_ref, s2_ref, s3_ref), subs):
        ref[...] = m
    c_ref[...] = jnp.maximum(jnp.maximum(subs[0], subs[1]),
                             jnp.maximum(subs[2], subs[3]))


def _theta_body(cm_ref, th_ref):
    w = cm_ref[...]
    iota = lax.broadcasted_iota(jnp.int32, w.shape, 1)
    for _ in range(_K):
        m = jnp.max(w, axis=1, keepdims=True)
        am = jnp.min(jnp.where(w == m, iota, _NCH), axis=1, keepdims=True)
        th_ref[...] = m
        w = jnp.where(iota == am, -jnp.inf, w)


def _sc_select_decode(acts_flat, cms, theta, W_dec, b_dec2):
    info = plsc.get_sparse_core_info()
    nw = info.num_cores * info.num_subcores
    tpw = _NP // nw
    mesh = plsc.VectorSubcoreMesh(core_axis_name="c", subcore_axis_name="s")
    ncand = _GCAP * _SUB

    @functools.partial(
        pl.kernel, mesh=mesh,
        out_type=jax.ShapeDtypeStruct((_NP, _D), jnp.float32),
        compiler_params=pltpu.CompilerParams(needs_layout_passes=False),
        scratch_types=[
            [pltpu.VMEM((tpw, _NCH), jnp.float32) for _ in range(_NSUBS)],
            pltpu.VMEM((tpw + 16,), jnp.float32),
            [pltpu.VMEM((96,), jnp.int32) for _ in range(2)],
            [pltpu.VMEM((_GCAP,), jnp.int32) for _ in range(2)],
            [pltpu.VMEM((_GCAP, _CH), jnp.float32) for _ in range(2)],
            pltpu.VMEM((ncand + 16,), jnp.float32),
            pltpu.VMEM((ncand + 16,), jnp.int32),
            [pltpu.VMEM((_K + 16,), jnp.float32) for _ in range(2)],
            [pltpu.VMEM((_K + 16,), jnp.int32) for _ in range(2)],
            [pltpu.VMEM((_K, _D), jnp.float32) for _ in range(2)],
            pltpu.VMEM((1, _D), jnp.float32),
            pltpu.VMEM((1, _D), jnp.float32),
            [pltpu.SemaphoreType.DMA for _ in range(2)],
            [pltpu.SemaphoreType.DMA for _ in range(2)],
        ])
    def sc_k(acts_hbm, cm0_hbm, cm1_hbm, cm2_hbm, cm3_hbm, th_hbm,
             wdec_hbm, bdec_hbm, out_hbm,
             cm_v, theta_v, subids, gidx, chunks,
             cand_val, cand_idx, sel_val, sel_idx,
             rows_v, bdec_v, acc_v, sg, sr):
        cc = lax.axis_index("c")
        ss = lax.axis_index("s")
        wid = ss * info.num_cores + cc
        base = wid * tpw
        for ref, hbm in zip(cm_v, (cm0_hbm, cm1_hbm, cm2_hbm, cm3_hbm)):
            pltpu.sync_copy(hbm.at[pl.ds(base, tpw), :], ref)
        pltpu.sync_copy(th_hbm.at[pl.ds(base, tpw)],
                        theta_v.at[pl.ds(0, tpw)])
        pltpu.sync_copy(bdec_hbm, bdec_v)
        lane = lax.broadcasted_iota(jnp.int32, (16,), 0)
        zero16 = jnp.zeros((16,), jnp.int32)
        for b in range(2):
            for l in range(6):
                subids[b][pl.ds(l * 16, 16)] = zero16

        def scan_issue(t, b):
            th_v = jnp.full((16,), theta_v[pl.ds(t, 16)][0], jnp.float32)
            nsub = jnp.int32(0)
            for s in range(_NSUBS):
                for cb in range(_NCH // 16):
                    v = cm_v[s][t, cb * 16:(cb + 1) * 16]
                    m = v >= th_v
                    idv = jnp.full((16,), s * _NCH + cb * 16, jnp.int32) + lane
                    mi = jnp.where(m, 1, 0).astype(jnp.int32)
                    pos = jnp.minimum(plsc.cumsum(mi) - 1 + nsub, 95)
                    plsc.store_scatter(subids[b], [pos], idv, mask=m)
                    nsub = nsub + plsc.all_reduce_population_count(m)[0]
            tok96 = jnp.full((16,), t * _NCH + base * _NCH, jnp.int32)
            for l in range(_GCAP // 16):
                sv = subids[b][pl.ds(l * 16, 16)]
                gidx[b][pl.ds(l * 16, 16)] = tok96 + sv % _NCH
            pltpu.async_copy(acts_hbm.at[gidx[b]], chunks[b], sg[b])
            return nsub

        def count_ge(vec, cnt, nv):
            def cbody(i, acc):
                v = cand_val[pl.ds(i * 16, 16)]
                mm = ((lane + i * 16) < cnt) & (v >= vec)
                return acc + plsc.all_reduce_population_count(mm)[0]
            return lax.fori_loop(0, nv, cbody, jnp.int32(0))

        def select_phase(t, b, nsub):
            pltpu.make_async_copy(acts_hbm.at[gidx[b]], chunks[b],
                                  sg[b]).wait()
            th = theta_v[pl.ds(t, 16)][0]
            th_v = jnp.full((16,), th, jnp.float32)
            inf_v = jnp.full((16,), jnp.inf, jnp.float32)
            svs = tuple(subids[b][pl.ds(q * 16, 16)]
                        for q in range(_UNROLL // 16))
            cnt = jnp.int32(0)

            def emit(i, sid, valid, cnt):
                s = sid // _NCH
                c = sid % _NCH
                off = s * _SUB
                fj = c * _CH + off
                th_i = jnp.where(valid, th_v, inf_v)
                for l in range(_SUB // 16):
                    v = chunks[b][i, pl.ds(off + l * 16, 16)]
                    m = v >= th_i
                    iv = jnp.full((16,), fj + l * 16, jnp.int32) + lane
                    mi = jnp.where(m, 1, 0).astype(jnp.int32)
                    pos = plsc.cumsum(mi) - 1 + cnt
                    plsc.store_scatter(cand_val, [pos], v, mask=m)
                    plsc.store_scatter(cand_idx, [pos], iv, mask=m)
                    cnt = cnt + plsc.all_reduce_population_count(m)[0]
                return cnt

            for i in range(_UNROLL):
                cnt = emit(i, svs[i // 16][i % 16], i < nsub, cnt)

            def tail(i, cnt):
                return emit(i, subids[b][pl.ds(i, 16)][0], True, cnt)

            cnt = lax.fori_loop(_UNROLL, jnp.minimum(nsub, _GCAP), tail, cnt)
            nv = (cnt + 15) // 16

            def count_ge3(vec, cnt):
                acc = jnp.int32(0)
                for q in range(3):
                    v = cand_val[q * 16:(q + 1) * 16]
                    mm = ((lane + q * 16) < cnt) & (v >= vec)
                    acc = acc + plsc.all_reduce_population_count(mm)[0]
                return acc

            def bsearch(counter):
                def sbody(_, lohi):
                    lo, hi = lohi
                    mid = lo + (hi - lo + 1) // 2
                    midf = plsc.bitcast(jnp.full((16,), mid, jnp.int32),
                                        jnp.float32)
                    big = counter(midf) >= _K
                    return (jnp.where(big, mid, lo),
                            jnp.where(big, hi, mid - 1))

                lo, _hi = lax.fori_loop(
                    0, 31, sbody, (jnp.int32(0), jnp.int32(0x7F800000)))
                gt = counter(plsc.bitcast(
                    jnp.full((16,), lo + 1, jnp.int32), jnp.float32))
                return lo, gt

            lo, n_gt = lax.cond(
                cnt <= 48,
                lambda: bsearch(lambda vec: count_ge3(vec, cnt)),
                lambda: bsearch(lambda vec: count_ge(vec, cnt, nv)))
            tstar = plsc.bitcast(jnp.full((16,), lo, jnp.int32), jnp.float32)
            need = _K - n_gt

            def selbody(i, carry):
                scnt, eqs = carry
                v = cand_val[pl.ds(i * 16, 16)]
                iv = cand_idx[pl.ds(i * 16, 16)]
                valid = (lane + i * 16) < cnt
                m_gt = valid & (v > tstar)
                m_eq = valid & (v == tstar)
                meqi = jnp.where(m_eq, 1, 0).astype(jnp.int32)
                rank = plsc.cumsum(meqi)
                sel = m_gt | (m_eq & ((rank + eqs) <= need))
                seli = jnp.where(sel, 1, 0).astype(jnp.int32)
                pos = plsc.cumsum(seli) - 1 + scnt
                plsc.store_scatter(sel_val[b], [pos], v, mask=sel)
                plsc.store_scatter(sel_idx[b], [pos], iv, mask=sel)
                return (scnt + plsc.all_reduce_population_count(sel)[0],
                        eqs + plsc.all_reduce_population_count(m_eq)[0])

            lax.fori_loop(0, nv, selbody, (jnp.int32(0), jnp.int32(0)))

            pltpu.async_copy(wdec_hbm.at[sel_idx[b].at[pl.ds(0, _K)]],
                             rows_v[b], sr[b])

        def decode_phase(t, b):
            pltpu.make_async_copy(wdec_hbm.at[sel_idx[b].at[pl.ds(0, _K)]],
                                  rows_v[b], sr[b]).wait()
            sv_row = (sel_val[b][0:16], sel_val[b][16:32])
            splats = tuple(
                jnp.full((16,), sv_row[j // 16][j % 16], jnp.float32)
                for j in range(_K))

            def gloop(g, carry):
                for h in range(2):
                    gof = (g * 2 + h) * 16
                    acc = bdec_v[0, pl.ds(gof, 16)]
                    for j in range(_K):
                        acc = acc + carry[j] * rows_v[b][j, pl.ds(gof, 16)]
                    acc_v[0, pl.ds(gof, 16)] = acc
                return carry

            lax.fori_loop(0, _D // 32, gloop, splats)
            pltpu.sync_copy(acc_v, out_hbm.at[pl.ds(base + t, 1), :])

        nsub_a0 = scan_issue(0, 0)

        def pair_body(i, nsub_a):
            ta = 2 * i
            nsub_b = scan_issue(ta + 1, 1)
            select_phase(ta, 0, nsub_a)
            nsub_n = scan_issue((ta + 2) % tpw, 0)
            select_phase(ta + 1, 1, nsub_b)
            decode_phase(ta, 0)
            decode_phase(ta + 1, 1)
            return nsub_n

        fin = lax.fori_loop(0, tpw // 2, pair_body, nsub_a0)
        _ = fin
        pltpu.make_async_copy(acts_hbm.at[gidx[0]], chunks[0], sg[0]).wait()

    return sc_k(acts_flat, *cms, theta, W_dec, b_dec2)


def kernel(x, W_enc, W_dec, b_dec):
    B, S, D = x.shape
    N = B * S
    xf = x.reshape(N, D)
    xf = jnp.concatenate(
        [xf, jnp.broadcast_to(xf[:1], (_NP - N, D))], axis=0)
    b2 = b_dec.reshape(1, D)

    acts3 = pl.pallas_call(
        _enc_body,
        grid=(_NP // _TB_MM, _F // _FB),
        in_specs=[
            pl.BlockSpec((_TB_MM, D), lambda t, f: (t, 0)),
            pl.BlockSpec((D, _FB), lambda t, f: (0, f)),
            pl.BlockSpec((1, D), lambda t, f: (0, 0)),
        ],
        out_specs=pl.BlockSpec((_TB_MM, _FB // _CH, _CH),
                               lambda t, f: (t, f, 0)),
        out_shape=jax.ShapeDtypeStruct((_NP, _NCH, _CH), jnp.float32),
    )(xf, W_enc.astype(jnp.bfloat16), b2)

    sub_spec = pl.BlockSpec((_TB_SEL, _NCH), lambda t: (t, 0))
    sub_shape = jax.ShapeDtypeStruct((_NP, _NCH), jnp.float32)
    *cms, cm128 = pl.pallas_call(
        _cmax_body,
        grid=(_NP // _TB_SEL,),
        in_specs=[pl.BlockSpec((_TB_SEL, _NCH, _CH), lambda t: (t, 0, 0))],
        out_specs=[sub_spec] * 5,
        out_shape=[sub_shape] * 5,
    )(acts3)

    theta = pl.pallas_call(
        _theta_body,
        grid=(_NP // _TB_SEL,),
        in_specs=[sub_spec],
        out_specs=pl.BlockSpec((_TB_SEL, 1), lambda t: (t, 0)),
        out_shape=jax.ShapeDtypeStruct((_NP, 1), jnp.float32),
    )(cm128)

    acts_flat = acts3.reshape(_NP * _NCH, _CH)
    xhat = _sc_select_decode(acts_flat, cms, theta.reshape(_NP), W_dec, b2)
    return xhat[:N].reshape(B, S, D)


# speedup vs baseline: 1.0264x; 1.0264x over previous
import functools

import jax
import jax.numpy as jnp
from jax import lax
from jax.experimental import pallas as pl
from jax.experimental.pallas import tpu as pltpu
from jax.experimental.pallas import tpu_sc as plsc

_D = 768
_F = 12288
_K = 32
_CH = 128
_NCH = _F // _CH
_SUB = 32
_NSUBS = 4
_NP = 3328
_TB_MM = 832
_FB = 1024
_TB_SEL = 208
_GCAP = 64
_UNROLL = 32


def _enc_body(x_ref, w_ref, b_ref, o_ref):
    xc = (x_ref[...] - b_ref[...]).astype(jnp.bfloat16)
    a = jnp.maximum(
        jnp.dot(xc, w_ref[...], preferred_element_type=jnp.float32), 0.0)
    o_ref[...] = a.reshape(a.shape[0], _FB // _CH, _CH)


def _cmax_body(a_ref, s0_ref, s1_ref, s2_ref, s3_ref, c_ref):
    a = a_ref[...]
    subs = [jnp.max(a[:, :, s * _SUB:(s + 1) * _SUB], axis=2)
            for s in range(_NSUBS)]
    for ref, m in zip((s0_ref, s1_ref, s2_ref, s3_ref), subs):
        ref[...] = m
    c_ref[...] = jnp.maximum(jnp.maximum(subs[0], subs[1]),
                             jnp.maximum(subs[2], subs[3]))


def _theta_body(cm_ref, th_ref):
    w = cm_ref[...]
    iota = lax.broadcasted_iota(jnp.int32, w.shape, 1)
    for _ in range(_K):
        m = jnp.max(w, axis=1, keepdims=True)
        am = jnp.min(jnp.where(w == m, iota, _NCH), axis=1, keepdims=True)
        th_ref[...] = m
        w = jnp.where(iota == am, -jnp.inf, w)


def _sc_select_decode(acts_flat, cms, theta, W_dec, b_dec2):
    info = plsc.get_sparse_core_info()
    nw = info.num_cores * info.num_subcores
    tpw = _NP // nw
    mesh = plsc.VectorSubcoreMesh(core_axis_name="c", subcore_axis_name="s")
    ncand = _GCAP * _SUB

    @functools.partial(
        pl.kernel, mesh=mesh,
        out_type=jax.ShapeDtypeStruct((_NP, _D), jnp.float32),
        compiler_params=pltpu.CompilerParams(needs_layout_passes=False),
        scratch_types=[
            [pltpu.VMEM((tpw, _NCH), jnp.float32) for _ in range(_NSUBS)],
            pltpu.VMEM((tpw + 16,), jnp.float32),
            [pltpu.VMEM((96,), jnp.int32) for _ in range(2)],
            [pltpu.VMEM((_GCAP,), jnp.int32) for _ in range(2)],
            [pltpu.VMEM((_GCAP, _CH), jnp.float32) for _ in range(2)],
            pltpu.VMEM((ncand + 16,), jnp.float32),
            pltpu.VMEM((ncand + 16,), jnp.int32),
            [pltpu.VMEM((_K + 16,), jnp.float32) for _ in range(2)],
            [pltpu.VMEM((_K + 16,), jnp.int32) for _ in range(2)],
            [pltpu.VMEM((_K, _D), jnp.float32) for _ in range(2)],
            pltpu.VMEM((1, _D), jnp.float32),
            pltpu.VMEM((1, _D), jnp.float32),
            [pltpu.SemaphoreType.DMA for _ in range(2)],
            [pltpu.SemaphoreType.DMA for _ in range(2)],
        ])
    def sc_k(acts_hbm, cm0_hbm, cm1_hbm, cm2_hbm, cm3_hbm, th_hbm,
             wdec_hbm, bdec_hbm, out_hbm,
             cm_v, theta_v, subids, gidx, chunks,
             cand_val, cand_idx, sel_val, sel_idx,
             rows_v, bdec_v, acc_v, sg, sr):
        cc = lax.axis_index("c")
        ss = lax.axis_index("s")
        wid = ss * info.num_cores + cc
        base = wid * tpw
        for ref, hbm in zip(cm_v, (cm0_hbm, cm1_hbm, cm2_hbm, cm3_hbm)):
            pltpu.sync_copy(hbm.at[pl.ds(base, tpw), :], ref)
        pltpu.sync_copy(th_hbm.at[pl.ds(base, tpw)],
                        theta_v.at[pl.ds(0, tpw)])
        pltpu.sync_copy(bdec_hbm, bdec_v)
        lane = lax.broadcasted_iota(jnp.int32, (16,), 0)
        zero16 = jnp.zeros((16,), jnp.int32)
        for b in range(2):
            for l in range(6):
                subids[b][pl.ds(l * 16, 16)] = zero16

        def scan_issue(t, b):
            th_v = jnp.full((16,), theta_v[pl.ds(t, 16)][0], jnp.float32)
            nsub = jnp.int32(0)
            for s in range(_NSUBS):
                for cb in range(_NCH // 16):
                    v = cm_v[s][t, cb * 16:(cb + 1) * 16]
                    m = v >= th_v
                    idv = jnp.full((16,), s * _NCH + cb * 16, jnp.int32) + lane
                    mi = jnp.where(m, 1, 0).astype(jnp.int32)
                    pos = jnp.minimum(plsc.cumsum(mi) - 1 + nsub, 95)
                    plsc.store_scatter(subids[b], [pos], idv, mask=m)
                    nsub = nsub + plsc.all_reduce_population_count(m)[0]
            tok96 = jnp.full((16,), t * _NCH + base * _NCH, jnp.int32)
            for l in range(_GCAP // 16):
                sv = subids[b][pl.ds(l * 16, 16)]
                gidx[b][pl.ds(l * 16, 16)] = tok96 + sv % _NCH
            pltpu.async_copy(acts_hbm.at[gidx[b]], chunks[b], sg[b])
            return nsub

        def count_ge(vec, cnt, nv):
            def cbody(i, acc):
                v = cand_val[pl.ds(i * 16, 16)]
                mm = ((lane + i * 16) < cnt) & (v >= vec)
                return acc + plsc.all_reduce_population_count(mm)[0]
            return lax.fori_loop(0, nv, cbody, jnp.int32(0))

        def select_phase(t, b, nsub):
            pltpu.make_async_copy(acts_hbm.at[gidx[b]], chunks[b],
                                  sg[b]).wait()
            th = theta_v[pl.ds(t, 16)][0]
            th_v = jnp.full((16,), th, jnp.float32)
            inf_v = jnp.full((16,), jnp.inf, jnp.float32)
            svs = tuple(subids[b][pl.ds(q * 16, 16)]
                        for q in range(_UNROLL // 16))
            cnt = jnp.int32(0)

            def emit(i, sid, valid, cnt):
                s = sid // _NCH
                c = sid % _NCH
                off = s * _SUB
                fj = c * _CH + off
                th_i = jnp.where(valid, th_v, inf_v)
                for l in range(_SUB // 16):
                    v = chunks[b][i, pl.ds(off + l * 16, 16)]
                    m = v >= th_i
                    iv = jnp.full((16,), fj + l * 16, jnp.int32) + lane
                    mi = jnp.where(m, 1, 0).astype(jnp.int32)
                    pos = plsc.cumsum(mi) - 1 + cnt
                    plsc.store_scatter(cand_val, [pos], v, mask=m)
                    plsc.store_scatter(cand_idx, [pos], iv, mask=m)
                    cnt = cnt + plsc.all_reduce_population_count(m)[0]
                return cnt

            for i in range(_UNROLL):
                cnt = emit(i, svs[i // 16][i % 16], i < nsub, cnt)

            def tail(i, cnt):
                return emit(i, subids[b][pl.ds(i, 16)][0], True, cnt)

            cnt = lax.fori_loop(_UNROLL, jnp.minimum(nsub, _GCAP), tail, cnt)
            nv = (cnt + 15) // 16

            def count_ge3(vec, cnt):
                acc = jnp.int32(0)
                for q in range(3):
                    v = cand_val[q * 16:(q + 1) * 16]
                    mm = ((lane + q * 16) < cnt) & (v >= vec)
                    acc = acc + plsc.all_reduce_population_count(mm)[0]
                return acc

            def bsearch(counter):
                def sbody(_, lohi):
                    lo, hi = lohi
                    mid = lo + (hi - lo + 1) // 2
                    midf = plsc.bitcast(jnp.full((16,), mid, jnp.int32),
                                        jnp.float32)
                    big = counter(midf) >= _K
                    return (jnp.where(big, mid, lo),
                            jnp.where(big, hi, mid - 1))

                lo, _hi = lax.fori_loop(
                    0, 31, sbody, (jnp.int32(0), jnp.int32(0x7F800000)))
                gt = counter(plsc.bitcast(
                    jnp.full((16,), lo + 1, jnp.int32), jnp.float32))
                return lo, gt

            lo, n_gt = lax.cond(
                cnt <= 48,
                lambda: bsearch(lambda vec: count_ge3(vec, cnt)),
                lambda: bsearch(lambda vec: count_ge(vec, cnt, nv)))
            tstar = plsc.bitcast(jnp.full((16,), lo, jnp.int32), jnp.float32)
            need = _K - n_gt

            def selbody(i, carry):
                scnt, eqs = carry
                v = cand_val[pl.ds(i * 16, 16)]
                iv = cand_idx[pl.ds(i * 16, 16)]
                valid = (lane + i * 16) < cnt
                m_gt = valid & (v > tstar)
                m_eq = valid & (v == tstar)
                meqi = jnp.where(m_eq, 1, 0).astype(jnp.int32)
                rank = plsc.cumsum(meqi)
                sel = m_gt | (m_eq & ((rank + eqs) <= need))
                seli = jnp.where(sel, 1, 0).astype(jnp.int32)
                pos = plsc.cumsum(seli) - 1 + scnt
                plsc.store_scatter(sel_val[b], [pos], v, mask=sel)
                plsc.store_scatter(sel_idx[b], [pos], iv, mask=sel)
                return (scnt + plsc.all_reduce_population_count(sel)[0],
                        eqs + plsc.all_reduce_population_count(m_eq)[0])

            lax.fori_loop(0, nv, selbody, (jnp.int32(0), jnp.int32(0)))

            pltpu.async_copy(wdec_hbm.at[sel_idx[b].at[pl.ds(0, _K)]],
                             rows_v[b], sr[b])

        def decode_phase(t, b):
            pltpu.make_async_copy(wdec_hbm.at[sel_idx[b].at[pl.ds(0, _K)]],
                                  rows_v[b], sr[b]).wait()
            sv_row = (sel_val[b][0:16], sel_val[b][16:32])
            splats = tuple(
                jnp.full((16,), sv_row[j // 16][j % 16], jnp.float32)
                for j in range(_K))

            def gloop(g, carry):
                acc = bdec_v[0, pl.ds(g * 16, 16)]
                for j in range(_K):
                    acc = acc + carry[j] * rows_v[b][j, pl.ds(g * 16, 16)]
                acc_v[0, pl.ds(g * 16, 16)] = acc
                return carry

            lax.fori_loop(0, _D // 16, gloop, splats)
            pltpu.sync_copy(acc_v, out_hbm.at[pl.ds(base + t, 1), :])

        nsub_a0 = scan_issue(0, 0)

        def pair_body(i, nsub_a):
            ta = 2 * i
            nsub_b = scan_issue(ta + 1, 1)
            select_phase(ta, 0, nsub_a)
            nsub_n = scan_issue((ta + 2) % tpw, 0)
            select_phase(ta + 1, 1, nsub_b)
            decode_phase(ta, 0)
            decode_phase(ta + 1, 1)
            return nsub_n

        fin = lax.fori_loop(0, tpw // 2, pair_body, nsub_a0)
        _ = fin
        pltpu.make_async_copy(acts_hbm.at[gidx[0]], chunks[0], sg[0]).wait()

    return sc_k(acts_flat, *cms, theta, W_dec, b_dec2)


def kernel(x, W_enc, W_dec, b_dec):
    B, S, D = x.shape
    N = B * S
    xf = x.reshape(N, D)
    xf = jnp.concatenate(
        [xf, jnp.broadcast_to(xf[:1], (_NP - N, D))], axis=0)
    b2 = b_dec.reshape(1, D)

    acts3 = pl.pallas_call(
        _enc_body,
        grid=(_NP // _TB_MM, _F // _FB),
        in_specs=[
            pl.BlockSpec((_TB_MM, D), lambda t, f: (t, 0)),
            pl.BlockSpec((D, _FB), lambda t, f: (0, f)),
            pl.BlockSpec((1, D), lambda t, f: (0, 0)),
        ],
        out_specs=pl.BlockSpec((_TB_MM, _FB // _CH, _CH),
                               lambda t, f: (t, f, 0)),
        out_shape=jax.ShapeDtypeStruct((_NP, _NCH, _CH), jnp.float32),
    )(xf, W_enc.astype(jnp.bfloat16), b2)

    sub_spec = pl.BlockSpec((_TB_SEL, _NCH), lambda t: (t, 0))
    sub_shape = jax.ShapeDtypeStruct((_NP, _NCH), jnp.float32)
    *cms, cm128 = pl.pallas_call(
        _cmax_body,
        grid=(_NP // _TB_SEL,),
        in_specs=[pl.BlockSpec((_TB_SEL, _NCH, _CH), lambda t: (t, 0, 0))],
        out_specs=[sub_spec] * 5,
        out_shape=[sub_shape] * 5,
    )(acts3)

    theta = pl.pallas_call(
        _theta_body,
        grid=(_NP // _TB_SEL,),
        in_specs=[sub_spec],
        out_specs=pl.BlockSpec((_TB_SEL, 1), lambda t: (t, 0)),
        out_shape=jax.ShapeDtypeStruct((_NP, 1), jnp.float32),
    )(cm128)

    acts_flat = acts3.reshape(_NP * _NCH, _CH)
    xhat = _sc_select_decode(acts_flat, cms, theta.reshape(_NP), W_dec, b2)
    return xhat[:N].reshape(B, S, D)
